# Initial kernel scaffold; baseline (speedup 1.0000x reference)
import functools

import jax
import jax.numpy as jnp
from jax import lax
from jax.experimental import pallas as pl
from jax.experimental.pallas import tpu as pltpu
from jax.experimental.pallas import tpu_sc as plsc

N_NODES = 10000
N_EDGES = 160000
D = 256
DH = D // 2

CHUNK = 80
EDGES_PER_TILE = N_EDGES // 16
N_CHUNKS = EDGES_PER_TILE // CHUNK
ROWS_PER_TILE = N_NODES // 16
OUT_CHUNK = 125


def _spmm_body(rows_hbm, cols_hbm, vals_hbm, xs_hbm, h_hbm,
               rows_v, cols_v, vals_v, buf, zbuf, acc, sem):
    c = lax.axis_index("c")
    s = lax.axis_index("s")

    @pl.loop(0, OUT_CHUNK)
    def _(r):
        zrow = zbuf.at[r]
        for j in range(DH // 16):
            zrow[pl.ds(16 * j, 16)] = jnp.zeros((16,), jnp.float32)

    @pl.loop(0, ROWS_PER_TILE // OUT_CHUNK)
    def _(i):
        pltpu.sync_copy(zbuf, acc.at[pl.ds(s * ROWS_PER_TILE + i * OUT_CHUNK,
                                           OUT_CHUNK)])

    plsc.subcore_barrier()

    coff = jnp.full((16,), c * N_NODES, jnp.int32)

    @pl.loop(0, N_CHUNKS)
    def _(k):
        base = s * EDGES_PER_TILE + k * CHUNK
        pltpu.sync_copy(rows_hbm.at[pl.ds(base, CHUNK)], rows_v)
        pltpu.sync_copy(cols_hbm.at[pl.ds(base, CHUNK)], cols_v)
        pltpu.sync_copy(vals_hbm.at[pl.ds(base, CHUNK)], vals_v)
        for j in range(CHUNK // 16):
            cols_v[pl.ds(16 * j, 16)] = cols_v[pl.ds(16 * j, 16)] + coff
        pltpu.async_copy(xs_hbm.at[cols_v], buf, sem).wait()

        @pl.loop(0, CHUNK)
        def _(e):
            valv = plsc.load_gather(vals_v, [jnp.full((16,), e, jnp.int32)])
            row = buf.at[e]
            for j in range(DH // 16):
                row[pl.ds(16 * j, 16)] = row[pl.ds(16 * j, 16)] * valv

        pltpu.sync_copy(buf, acc.at[rows_v], add=True)

    plsc.subcore_barrier()

    @pl.loop(0, ROWS_PER_TILE // OUT_CHUNK)
    def _(i):
        r0 = s * ROWS_PER_TILE + i * OUT_CHUNK
        pltpu.sync_copy(acc.at[pl.ds(r0, OUT_CHUNK)], zbuf)
        pltpu.sync_copy(zbuf, h_hbm.at[pl.ds(c * N_NODES + r0, OUT_CHUNK)])


def _spmm(rows, cols, vals, xs):
    mesh = plsc.VectorSubcoreMesh(core_axis_name="c", subcore_axis_name="s")
    return pl.kernel(
        _spmm_body,
        out_type=jax.ShapeDtypeStruct((2 * N_NODES, DH), jnp.float32),
        mesh=mesh,
        scratch_types=[
            pltpu.VMEM((CHUNK,), jnp.int32),
            pltpu.VMEM((CHUNK,), jnp.int32),
            pltpu.VMEM((CHUNK,), jnp.float32),
            pltpu.VMEM((CHUNK, DH), jnp.float32),
            pltpu.VMEM((OUT_CHUNK, DH), jnp.float32),
            pltpu.VMEM_SHARED((N_NODES, DH), jnp.float32),
            pltpu.SemaphoreType.DMA,
        ],
        name="spmm_sc",
    )(rows, cols, vals, xs)


BM = 400


def _linear_body(h0_ref, h1_ref, a_ref, b_ref, o_ref):
    acc = jnp.dot(h0_ref[...], a_ref[0:DH, :],
                  preferred_element_type=jnp.float32)
    acc = acc + jnp.dot(h1_ref[...], a_ref[DH:D, :],
                        preferred_element_type=jnp.float32)
    o_ref[...] = jnp.maximum(acc + b_ref[...], 0.0)


def _linear_relu(h_stacked, a, b2):
    nb = N_NODES // BM
    return pl.pallas_call(
        _linear_body,
        grid=(nb,),
        in_specs=[
            pl.BlockSpec((BM, DH), lambda i: (i, 0)),
            pl.BlockSpec((BM, DH), lambda i, nb=nb: (i + nb, 0)),
            pl.BlockSpec((D, D), lambda i: (0, 0)),
            pl.BlockSpec((1, D), lambda i: (0, 0)),
        ],
        out_specs=pl.BlockSpec((BM, D), lambda i: (i, 0)),
        out_shape=jax.ShapeDtypeStruct((N_NODES, D), jnp.float32),
    )(h_stacked, h_stacked, a, b2)


@jax.jit
def kernel(x, adj_indices, adj_values, W, b):
    rows = adj_indices[0].astype(jnp.int32)
    cols = adj_indices[1].astype(jnp.int32)
    xs = jnp.concatenate([x[:, :DH], x[:, DH:]], axis=0)
    h_stacked = _spmm(rows, cols, adj_values, xs)
    return _linear_relu(h_stacked, W.T, b.reshape(1, D))


# speedup vs baseline: 2.8653x; 2.8653x over previous
import functools

import jax
import jax.numpy as jnp
from jax import lax
from jax.experimental import pallas as pl
from jax.experimental.pallas import tpu as pltpu
from jax.experimental.pallas import tpu_sc as plsc

N_NODES = 10000
N_EDGES = 160000
D = 256
DH = D // 2

CHUNK = 80
EDGES_PER_TILE = N_EDGES // 16
N_CHUNKS = EDGES_PER_TILE // CHUNK
OUT_TILE = 624
OUT_CHUNK = 208
OUT_TAIL = N_NODES - 16 * OUT_TILE
INTERP = False


def _spmm_body(rows_hbm, cols_hbm, vals_hbm, xs_hbm, h_hbm,
               rows_v, cols_v, vals_v, buf, zbuf, acc, sem):
    c = lax.axis_index("c")
    s = lax.axis_index("s")

    @pl.loop(0, OUT_CHUNK)
    def _(r):
        zrow = zbuf.at[r]
        for j in range(DH // 16):
            zrow[pl.ds(16 * j, 16)] = jnp.zeros((16,), jnp.float32)

    @pl.loop(0, OUT_TILE // OUT_CHUNK)
    def _(i):
        pltpu.sync_copy(zbuf, acc.at[pl.ds(s * OUT_TILE + i * OUT_CHUNK,
                                           OUT_CHUNK)])

    @pl.when(s == 0)
    def _():
        pltpu.sync_copy(zbuf.at[pl.ds(0, OUT_TAIL)],
                        acc.at[pl.ds(16 * OUT_TILE, OUT_TAIL)])

    plsc.subcore_barrier()

    coff = jnp.full((16,), c * N_NODES, jnp.int32)

    @pl.loop(0, N_CHUNKS)
    def _(k):
        base = s * EDGES_PER_TILE + k * CHUNK
        pltpu.sync_copy(rows_hbm.at[pl.ds(base, CHUNK)], rows_v)
        pltpu.sync_copy(cols_hbm.at[pl.ds(base, CHUNK)], cols_v)
        pltpu.sync_copy(vals_hbm.at[pl.ds(base, CHUNK)], vals_v)
        for j in range(CHUNK // 16):
            cols_v[pl.ds(16 * j, 16)] = cols_v[pl.ds(16 * j, 16)] + coff
        pltpu.async_copy(xs_hbm.at[cols_v], buf, sem).wait()

        @pl.loop(0, CHUNK // 16)
        def _(g):
            v16 = vals_v[pl.ds(g * 16, 16)]
            for el in range(16):
                valv = jnp.full((16,), v16[el], jnp.float32)
                row = buf.at[g * 16 + el]
                for j in range(DH // 16):
                    row[pl.ds(16 * j, 16)] = row[pl.ds(16 * j, 16)] * valv

        pltpu.sync_copy(buf, acc.at[rows_v], add=True)

    plsc.subcore_barrier()

    @pl.loop(0, OUT_TILE // OUT_CHUNK)
    def _(i):
        r0 = s * OUT_TILE + i * OUT_CHUNK
        pltpu.sync_copy(acc.at[pl.ds(r0, OUT_CHUNK)], zbuf)
        pltpu.sync_copy(zbuf, h_hbm.at[pl.ds(c * N_NODES + r0, OUT_CHUNK)])

    @pl.when(s == 0)
    def _():
        r0 = 16 * OUT_TILE
        pltpu.sync_copy(acc.at[pl.ds(r0, OUT_TAIL)], zbuf.at[pl.ds(0, OUT_TAIL)])
        pltpu.sync_copy(zbuf.at[pl.ds(0, OUT_TAIL)],
                        h_hbm.at[pl.ds(c * N_NODES + r0, OUT_TAIL)])


def _spmm(rows, cols, vals, xs):
    mesh = plsc.VectorSubcoreMesh(core_axis_name="c", subcore_axis_name="s",
                                  num_cores=2, num_subcores=16)
    return pl.kernel(
        _spmm_body,
        out_type=jax.ShapeDtypeStruct((2 * N_NODES, DH), jnp.float32),
        mesh=mesh,
        scratch_types=[
            pltpu.VMEM((CHUNK,), jnp.int32),
            pltpu.VMEM((CHUNK,), jnp.int32),
            pltpu.VMEM((CHUNK,), jnp.float32),
            pltpu.VMEM((CHUNK, DH), jnp.float32),
            pltpu.VMEM((OUT_CHUNK, DH), jnp.float32),
            pltpu.VMEM_SHARED((N_NODES, DH), jnp.float32),
            pltpu.SemaphoreType.DMA,
        ],
        name="spmm_sc",
        interpret=INTERP,
    )(rows, cols, vals, xs)


BM = 400


def _linear_body(h0_ref, h1_ref, a_ref, b_ref, o_ref):
    acc = jnp.dot(h0_ref[...], a_ref[0:DH, :],
                  preferred_element_type=jnp.float32)
    acc = acc + jnp.dot(h1_ref[...], a_ref[DH:D, :],
                        preferred_element_type=jnp.float32)
    o_ref[...] = jnp.maximum(acc + b_ref[...], 0.0)


def _linear_relu(h_stacked, a, b2):
    nb = N_NODES // BM
    return pl.pallas_call(
        _linear_body,
        grid=(nb,),
        in_specs=[
            pl.BlockSpec((BM, DH), lambda i: (i, 0)),
            pl.BlockSpec((BM, DH), lambda i, nb=nb: (i + nb, 0)),
            pl.BlockSpec((D, D), lambda i: (0, 0)),
            pl.BlockSpec((1, D), lambda i: (0, 0)),
        ],
        out_specs=pl.BlockSpec((BM, D), lambda i: (i, 0)),
        out_shape=jax.ShapeDtypeStruct((N_NODES, D), jnp.float32),
        interpret=INTERP,
    )(h_stacked, h_stacked, a, b2)


@jax.jit
def kernel(x, adj_indices, adj_values, W, b):
    rows = adj_indices[0].astype(jnp.int32)
    cols = adj_indices[1].astype(jnp.int32)
    xs = jnp.concatenate([x[:, :DH], x[:, DH:]], axis=0)
    h_stacked = _spmm(rows, cols, adj_values, xs)
    return _linear_relu(h_stacked, W.T, b.reshape(1, D))


# speedup vs baseline: 6.2983x; 2.1981x over previous
import functools

import jax
import jax.numpy as jnp
from jax import lax
from jax.experimental import pallas as pl
from jax.experimental.pallas import tpu as pltpu
from jax.experimental.pallas import tpu_sc as plsc

N_NODES = 10000
N_EDGES = 160000
D = 256
DH = D // 2

CHUNK = 80
EDGES_PER_TILE = N_EDGES // 16
N_CHUNKS = EDGES_PER_TILE // CHUNK
BLOCK = 5
N_BLOCKS = N_CHUNKS // BLOCK
NBUF = 3
OUT_TILE = 624
OUT_CHUNK = 104
OUT_TAIL = N_NODES - 16 * OUT_TILE
INTERP = False


def _spmm_body(edata_hbm, xs_hbm, h_hbm,
               blk,
               cidx0, rows0, rvals0, buf0,
               cidx1, rows1, rvals1, buf1,
               cidx2, rows2, rvals2, buf2,
               zbuf, acc,
               gsem0, gsem1, gsem2, ssem0, ssem1, ssem2):
    c = lax.axis_index("c")
    s = lax.axis_index("s")

    @pl.loop(0, OUT_CHUNK)
    def _(r):
        zrow = zbuf.at[r]
        for j in range(DH // 16):
            zrow[pl.ds(16 * j, 16)] = jnp.zeros((16,), jnp.float32)

    @pl.loop(0, OUT_TILE // OUT_CHUNK)
    def _(i):
        pltpu.sync_copy(zbuf, acc.at[pl.ds(s * OUT_TILE + i * OUT_CHUNK,
                                           OUT_CHUNK)])

    @pl.when(s == 0)
    def _():
        pltpu.sync_copy(zbuf.at[pl.ds(0, OUT_TAIL)],
                        acc.at[pl.ds(16 * OUT_TILE, OUT_TAIL)])

    plsc.subcore_barrier()

    coff = jnp.full((16,), c * N_NODES, jnp.int32)
    blk_len = 3 * BLOCK * CHUNK

    def load_block(b):
        base = (s * N_BLOCKS + b) * blk_len
        pltpu.sync_copy(edata_hbm.at[pl.ds(base, blk_len)], blk)

    def issue(k, cidx, rows_v, rvals, buf, gsem, ssem, first):
        if not first:
            pltpu.make_async_copy(buf, acc.at[rows_v], ssem).wait()
        b = k // BLOCK
        sub = k % BLOCK
        if isinstance(k, int):
            if sub == 0:
                load_block(b)
        else:
            @pl.when(sub == 0)
            def _():
                load_block(b)
        off = sub * CHUNK
        for j in range(CHUNK // 16):
            d = pl.ds(16 * j, 16)
            rows_v[d] = blk[pl.ds(off + 16 * j, 16)]
            cidx[d] = blk[pl.ds(BLOCK * CHUNK + off + 16 * j, 16)] + coff
            rvals[d] = plsc.bitcast(
                blk[pl.ds(2 * BLOCK * CHUNK + off + 16 * j, 16)], jnp.float32)
        pltpu.async_copy(xs_hbm.at[cidx], buf, gsem)

    def process(k, cidx, rows_v, rvals, buf, gsem, ssem):
        pltpu.make_async_copy(xs_hbm.at[cidx], buf, gsem).wait()

        @pl.loop(0, CHUNK // 16)
        def _(g):
            v16 = rvals[pl.ds(g * 16, 16)]
            for el in range(16):
                valv = jnp.full((16,), v16[el], jnp.float32)
                row = buf.at[g * 16 + el]
                for j in range(DH // 16):
                    row[pl.ds(16 * j, 16)] = row[pl.ds(16 * j, 16)] * valv

        pltpu.async_copy(buf, acc.at[rows_v], ssem, add=True)

    sets = [
        (cidx0, rows0, rvals0, buf0, gsem0, ssem0),
        (cidx1, rows1, rvals1, buf1, gsem1, ssem1),
        (cidx2, rows2, rvals2, buf2, gsem2, ssem2),
    ]

    issue(0, *sets[0], True)
    issue(1, *sets[1], True)
    process(0, *sets[0])
    issue(2, *sets[2], True)
    process(1, *sets[1])
    issue(3, *sets[0], False)
    process(2, *sets[2])

    @pl.loop(1, (N_CHUNKS - 2) // NBUF)
    def _(i):
        for j in range(NBUF):
            k = NBUF * i + j
            issue(k + 1, *sets[(j + 1) % NBUF], False)
            process(k, *sets[j])

    issue(N_CHUNKS - 1, *sets[(N_CHUNKS - 1) % NBUF], False)
    process(N_CHUNKS - 2, *sets[(N_CHUNKS - 2) % NBUF])
    process(N_CHUNKS - 1, *sets[(N_CHUNKS - 1) % NBUF])

    for j in range(NBUF):
        cidx, rows_v, rvals, buf, gsem, ssem = sets[j]
        pltpu.make_async_copy(buf, acc.at[rows_v], ssem).wait()

    plsc.subcore_barrier()

    @pl.loop(0, OUT_TILE // OUT_CHUNK)
    def _(i):
        r0 = s * OUT_TILE + i * OUT_CHUNK
        pltpu.sync_copy(acc.at[pl.ds(r0, OUT_CHUNK)], zbuf)
        pltpu.sync_copy(zbuf, h_hbm.at[pl.ds(c * N_NODES + r0, OUT_CHUNK)])

    @pl.when(s == 0)
    def _():
        r0 = 16 * OUT_TILE
        pltpu.sync_copy(acc.at[pl.ds(r0, OUT_TAIL)], zbuf.at[pl.ds(0, OUT_TAIL)])
        pltpu.sync_copy(zbuf.at[pl.ds(0, OUT_TAIL)],
                        h_hbm.at[pl.ds(c * N_NODES + r0, OUT_TAIL)])


def _chunk_scratch():
    return [
        pltpu.VMEM((CHUNK,), jnp.int32),
        pltpu.VMEM((CHUNK,), jnp.int32),
        pltpu.VMEM((CHUNK,), jnp.float32),
        pltpu.VMEM((CHUNK, DH), jnp.float32),
    ]


def _spmm(edata, xs):
    mesh = plsc.VectorSubcoreMesh(core_axis_name="c", subcore_axis_name="s",
                                  num_cores=2, num_subcores=16)
    return pl.kernel(
        _spmm_body,
        out_type=jax.ShapeDtypeStruct((2 * N_NODES, DH), jnp.float32),
        mesh=mesh,
        scratch_types=(
            [pltpu.VMEM((3 * BLOCK * CHUNK,), jnp.int32)]
            + _chunk_scratch() + _chunk_scratch() + _chunk_scratch()
            + [
                pltpu.VMEM((OUT_CHUNK, DH), jnp.float32),
                pltpu.VMEM_SHARED((N_NODES, DH), jnp.float32),
            ]
            + [pltpu.SemaphoreType.DMA] * 6
        ),
        name="spmm_sc",
        compiler_params=pltpu.CompilerParams(needs_layout_passes=False),
        interpret=INTERP,
    )(edata, xs)


BM = 400


def _linear_body(h0_ref, h1_ref, a_ref, b_ref, o_ref):
    acc = jnp.dot(h0_ref[...], a_ref[0:DH, :],
                  preferred_element_type=jnp.float32)
    acc = acc + jnp.dot(h1_ref[...], a_ref[DH:D, :],
                        preferred_element_type=jnp.float32)
    o_ref[...] = jnp.maximum(acc + b_ref[...], 0.0)


def _linear_relu(h_stacked, a, b2):
    nb = N_NODES // BM
    return pl.pallas_call(
        _linear_body,
        grid=(nb,),
        in_specs=[
            pl.BlockSpec((BM, DH), lambda i: (i, 0)),
            pl.BlockSpec((BM, DH), lambda i, nb=nb: (i + nb, 0)),
            pl.BlockSpec((D, D), lambda i: (0, 0)),
            pl.BlockSpec((1, D), lambda i: (0, 0)),
        ],
        out_specs=pl.BlockSpec((BM, D), lambda i: (i, 0)),
        out_shape=jax.ShapeDtypeStruct((N_NODES, D), jnp.float32),
        interpret=INTERP,
    )(h_stacked, h_stacked, a, b2)


@jax.jit
def kernel(x, adj_indices, adj_values, W, b):
    rows = adj_indices[0].astype(jnp.int32)
    cols = adj_indices[1].astype(jnp.int32)
    vals_i = lax.bitcast_convert_type(adj_values, jnp.int32)
    nb = 16 * N_BLOCKS
    edata = jnp.stack([rows.reshape(nb, BLOCK * CHUNK),
                       cols.reshape(nb, BLOCK * CHUNK),
                       vals_i.reshape(nb, BLOCK * CHUNK)], axis=1).reshape(-1)
    xs = jnp.concatenate([x[:, :DH], x[:, DH:]], axis=0)
    h_stacked = _spmm(edata, xs)
    return _linear_relu(h_stacked, W.T, b.reshape(1, D))


# speedup vs baseline: 8.2938x; 1.3168x over previous
import jax
import jax.numpy as jnp
from jax import lax
from jax.experimental import pallas as pl
from jax.experimental.pallas import tpu as pltpu
from jax.experimental.pallas import tpu_sc as plsc

N_NODES = 10000
N_EDGES = 160000
D = 256
DH = D // 2

CHUNK = 80
EDGES_PER_TILE = N_EDGES // 16
N_CHUNKS = EDGES_PER_TILE // CHUNK
BLOCK = 5
N_BLOCKS = N_CHUNKS // BLOCK
BLK_LEN = BLOCK * CHUNK
BLK_PAD = 512
NBUF = 4
OUT_TILE = 624
OUT_TAIL = N_NODES - 16 * OUT_TILE
OUT_REM = OUT_TILE - (OUT_TILE // CHUNK) * CHUNK


def _spmm_body(rows_hbm, cols_hbm, vals_hbm, xs_hbm, h_hbm,
               rblk, cblk, vblk,
               cidx0, rows0, rvals0, buf0,
               cidx1, rows1, rvals1, buf1,
               cidx2, rows2, rvals2, buf2,
               cidx3, rows3, rvals3, buf3,
               acc,
               bsem, gsem0, gsem1, gsem2, gsem3,
               ssem0, ssem1, ssem2, ssem3):
    c = lax.axis_index("c")
    s = lax.axis_index("s")

    @pl.loop(0, CHUNK)
    def _(r):
        zrow = buf0.at[r]
        for j in range(DH // 16):
            zrow[pl.ds(16 * j, 16)] = jnp.zeros((16,), jnp.float32)

    @pl.loop(0, OUT_TILE // CHUNK)
    def _(i):
        pltpu.sync_copy(buf0, acc.at[pl.ds(s * OUT_TILE + i * CHUNK, CHUNK)])

    pltpu.sync_copy(buf0.at[pl.ds(0, OUT_REM)],
                    acc.at[pl.ds(s * OUT_TILE + OUT_TILE - OUT_REM, OUT_REM)])

    @pl.when(s == 0)
    def _():
        pltpu.sync_copy(buf0.at[pl.ds(0, OUT_TAIL)],
                        acc.at[pl.ds(16 * OUT_TILE, OUT_TAIL)])

    plsc.subcore_barrier()

    coff = jnp.full((16,), c * N_NODES, jnp.int32)

    def start_block_load(b, slot):
        base = s * EDGES_PER_TILE + b * BLK_LEN
        dst = pl.ds(slot * BLK_PAD, BLK_LEN)
        pltpu.async_copy(rows_hbm.at[pl.ds(base, BLK_LEN)], rblk.at[dst], bsem)
        pltpu.async_copy(cols_hbm.at[pl.ds(base, BLK_LEN)], cblk.at[dst], bsem)
        pltpu.async_copy(vals_hbm.at[pl.ds(base, BLK_LEN)], vblk.at[dst], bsem)

    def wait_block(b, slot):
        base = s * EDGES_PER_TILE + b * BLK_LEN
        dst = pl.ds(slot * BLK_PAD, BLK_LEN)
        pltpu.make_async_copy(rows_hbm.at[pl.ds(base, BLK_LEN)], rblk.at[dst],
                              bsem).wait()
        pltpu.make_async_copy(cols_hbm.at[pl.ds(base, BLK_LEN)], cblk.at[dst],
                              bsem).wait()
        pltpu.make_async_copy(vals_hbm.at[pl.ds(base, BLK_LEN)], vblk.at[dst],
                              bsem).wait()

    def issue(k, cidx, rows_v, rvals, buf, gsem, ssem, first):
        if not first:
            pltpu.make_async_copy(buf, acc.at[rows_v], ssem).wait()
        b = k // BLOCK
        sub = k % BLOCK
        p = b % 2

        def rotate_blocks():
            wait_block(b, p)
            if isinstance(b, int):
                if b < N_BLOCKS - 1:
                    start_block_load(b + 1, 1 - p)
            else:
                @pl.when(b < N_BLOCKS - 1)
                def _():
                    start_block_load(b + 1, 1 - p)

        if isinstance(k, int):
            if sub == 0:
                rotate_blocks()
        else:
            @pl.when(sub == 0)
            def _():
                rotate_blocks()

        off = p * BLK_PAD + sub * CHUNK
        for j in range(CHUNK // 16):
            d = pl.ds(16 * j, 16)
            sl = pl.ds(off + 16 * j, 16)
            rows_v[d] = rblk[sl]
            cidx[d] = cblk[sl] + coff
            rvals[d] = vblk[sl]
        pltpu.async_copy(xs_hbm.at[cidx], buf, gsem)

    def process(k, cidx, rows_v, rvals, buf, gsem, ssem):
        pltpu.make_async_copy(xs_hbm.at[cidx], buf, gsem).wait()

        @pl.loop(0, CHUNK // 16)
        def _(g):
            v16 = rvals[pl.ds(g * 16, 16)]
            for el in range(16):
                valv = jnp.full((16,), v16[el], jnp.float32)
                row = buf.at[g * 16 + el]
                for j in range(DH // 16):
                    row[pl.ds(16 * j, 16)] = row[pl.ds(16 * j, 16)] * valv

        pltpu.async_copy(buf, acc.at[rows_v], ssem, add=True)

    sets = [
        (cidx0, rows0, rvals0, buf0, gsem0, ssem0),
        (cidx1, rows1, rvals1, buf1, gsem1, ssem1),
        (cidx2, rows2, rvals2, buf2, gsem2, ssem2),
        (cidx3, rows3, rvals3, buf3, gsem3, ssem3),
    ]

    start_block_load(0, 0)
    issue(0, *sets[0], True)
    issue(1, *sets[1], True)
    issue(2, *sets[2], True)
    process(0, *sets[0])
    issue(3, *sets[3], True)
    process(1, *sets[1])
    issue(4, *sets[0], False)
    process(2, *sets[2])
    issue(5, *sets[1], False)
    process(3, *sets[3])

    n_main = (N_CHUNKS - 1) // NBUF

    @pl.loop(1, n_main)
    def _(i):
        for j in range(NBUF):
            k = NBUF * i + j
            if j < NBUF - 1:
                issue(k + 2, *sets[(j + 2) % NBUF], False)
            else:
                @pl.when(i < n_main - 1)
                def _():
                    issue(k + 2, *sets[(j + 2) % NBUF], False)
            process(k, *sets[j])

    process(N_CHUNKS - 1, *sets[(N_CHUNKS - 1) % NBUF])

    for j in range(NBUF):
        cidx, rows_v, rvals, buf, gsem, ssem = sets[j]
        pltpu.make_async_copy(buf, acc.at[rows_v], ssem).wait()

    plsc.subcore_barrier()

    @pl.loop(0, OUT_TILE // CHUNK)
    def _(i):
        r0 = s * OUT_TILE + i * CHUNK
        pltpu.sync_copy(acc.at[pl.ds(r0, CHUNK)],
                        h_hbm.at[pl.ds(c * N_NODES + r0, CHUNK)])

    r1 = s * OUT_TILE + OUT_TILE - OUT_REM
    pltpu.sync_copy(acc.at[pl.ds(r1, OUT_REM)],
                    h_hbm.at[pl.ds(c * N_NODES + r1, OUT_REM)])

    @pl.when(s == 0)
    def _():
        r2 = 16 * OUT_TILE
        pltpu.sync_copy(acc.at[pl.ds(r2, OUT_TAIL)],
                        h_hbm.at[pl.ds(c * N_NODES + r2, OUT_TAIL)])


def _chunk_scratch():
    return [
        pltpu.VMEM((CHUNK,), jnp.int32),
        pltpu.VMEM((CHUNK,), jnp.int32),
        pltpu.VMEM((CHUNK,), jnp.float32),
        pltpu.VMEM((CHUNK, DH), jnp.float32),
    ]


def _spmm(rows, cols, vals, xs):
    mesh = plsc.VectorSubcoreMesh(core_axis_name="c", subcore_axis_name="s",
                                  num_cores=2, num_subcores=16)
    return pl.kernel(
        _spmm_body,
        out_type=jax.ShapeDtypeStruct((2 * N_NODES, DH), jnp.float32),
        mesh=mesh,
        scratch_types=(
            [
                pltpu.VMEM((2 * BLK_PAD,), jnp.int32),
                pltpu.VMEM((2 * BLK_PAD,), jnp.int32),
                pltpu.VMEM((2 * BLK_PAD,), jnp.float32),
            ]
            + _chunk_scratch() + _chunk_scratch()
            + _chunk_scratch() + _chunk_scratch()
            + [pltpu.VMEM_SHARED((N_NODES, DH), jnp.float32)]
            + [pltpu.SemaphoreType.DMA] * 9
        ),
        name="spmm_sc",
        compiler_params=pltpu.CompilerParams(needs_layout_passes=False),
    )(rows, cols, vals, xs)


BM = 2000


def _linear_body(h0_ref, h1_ref, a_ref, b_ref, o_ref):
    acc = jnp.dot(h0_ref[...], a_ref[0:DH, :],
                  preferred_element_type=jnp.float32)
    acc = acc + jnp.dot(h1_ref[...], a_ref[DH:D, :],
                        preferred_element_type=jnp.float32)
    o_ref[...] = jnp.maximum(acc + b_ref[...], 0.0)


def _linear_relu(h_stacked, a, b2):
    nb = N_NODES // BM
    return pl.pallas_call(
        _linear_body,
        grid=(nb,),
        in_specs=[
            pl.BlockSpec((BM, DH), lambda i: (i, 0)),
            pl.BlockSpec((BM, DH), lambda i, nb=nb: (i + nb, 0)),
            pl.BlockSpec((D, D), lambda i: (0, 0)),
            pl.BlockSpec((1, D), lambda i: (0, 0)),
        ],
        out_specs=pl.BlockSpec((BM, D), lambda i: (i, 0)),
        out_shape=jax.ShapeDtypeStruct((N_NODES, D), jnp.float32),
    )(h_stacked, h_stacked, a, b2)


@jax.jit
def kernel(x, adj_indices, adj_values, W, b):
    rows = adj_indices[0].astype(jnp.int32)
    cols = adj_indices[1].astype(jnp.int32)
    xs = jnp.concatenate([x[:, :DH], x[:, DH:]], axis=0)
    h_stacked = _spmm(rows, cols, adj_values, xs)
    return _linear_relu(h_stacked, W.T, b.reshape(1, D))


# speedup vs baseline: 8.3587x; 1.0078x over previous
import jax
import jax.numpy as jnp
from jax import lax
from jax.experimental import pallas as pl
from jax.experimental.pallas import tpu as pltpu
from jax.experimental.pallas import tpu_sc as plsc

N_NODES = 10000
N_EDGES = 160000
D = 256
DH = D // 2

CHUNK = 80
EDGES_PER_TILE = N_EDGES // 16
N_CHUNKS = EDGES_PER_TILE // CHUNK
BLOCK = 5
N_BLOCKS = N_CHUNKS // BLOCK
BLK_LEN = BLOCK * CHUNK
BLK_PAD = 512
NBUF = 4
OUT_TILE = 624
OUT_TAIL = N_NODES - 16 * OUT_TILE
OUT_REM = OUT_TILE - (OUT_TILE // CHUNK) * CHUNK


def _spmm_body(rows_hbm, cols_hbm, vals_hbm, xs_hbm, h_hbm,
               rblk, cblk, vblk,
               cidx0, rows0, rvals0, buf0,
               cidx1, rows1, rvals1, buf1,
               cidx2, rows2, rvals2, buf2,
               cidx3, rows3, rvals3, buf3,
               acc,
               bsem, gsem0, gsem1, gsem2, gsem3,
               ssem0, ssem1, ssem2, ssem3):
    c = lax.axis_index("c")
    s = lax.axis_index("s")

    @pl.loop(0, CHUNK)
    def _(r):
        zrow = buf0.at[r]
        for j in range(DH // 16):
            zrow[pl.ds(16 * j, 16)] = jnp.zeros((16,), jnp.float32)

    @pl.loop(0, OUT_TILE // CHUNK)
    def _(i):
        pltpu.sync_copy(buf0, acc.at[pl.ds(s * OUT_TILE + i * CHUNK, CHUNK)])

    pltpu.sync_copy(buf0.at[pl.ds(0, OUT_REM)],
                    acc.at[pl.ds(s * OUT_TILE + OUT_TILE - OUT_REM, OUT_REM)])

    @pl.when(s == 0)
    def _():
        pltpu.sync_copy(buf0.at[pl.ds(0, OUT_TAIL)],
                        acc.at[pl.ds(16 * OUT_TILE, OUT_TAIL)])

    plsc.subcore_barrier()

    pass

    def start_block_load(b, slot):
        base = s * EDGES_PER_TILE + b * BLK_LEN
        dst = pl.ds(slot * BLK_PAD, BLK_LEN)
        pltpu.async_copy(rows_hbm.at[pl.ds(base, BLK_LEN)], rblk.at[dst], bsem)
        pltpu.async_copy(cols_hbm.at[pl.ds(base, BLK_LEN)], cblk.at[dst], bsem)
        pltpu.async_copy(vals_hbm.at[pl.ds(base, BLK_LEN)], vblk.at[dst], bsem)

    def wait_block(b, slot):
        base = s * EDGES_PER_TILE + b * BLK_LEN
        dst = pl.ds(slot * BLK_PAD, BLK_LEN)
        pltpu.make_async_copy(rows_hbm.at[pl.ds(base, BLK_LEN)], rblk.at[dst],
                              bsem).wait()
        pltpu.make_async_copy(cols_hbm.at[pl.ds(base, BLK_LEN)], cblk.at[dst],
                              bsem).wait()
        pltpu.make_async_copy(vals_hbm.at[pl.ds(base, BLK_LEN)], vblk.at[dst],
                              bsem).wait()

    def issue(k, cidx, rows_v, rvals, buf, gsem, ssem, first):
        if not first:
            pltpu.make_async_copy(buf, acc.at[rows_v], ssem).wait()
        b = k // BLOCK
        sub = k % BLOCK
        p = b % 2

        def rotate_blocks():
            wait_block(b, p)
            if isinstance(b, int):
                if b < N_BLOCKS - 1:
                    start_block_load(b + 1, 1 - p)
            else:
                @pl.when(b < N_BLOCKS - 1)
                def _():
                    start_block_load(b + 1, 1 - p)

        if isinstance(k, int):
            if sub == 0:
                rotate_blocks()
        else:
            @pl.when(sub == 0)
            def _():
                rotate_blocks()

        off = p * BLK_PAD + sub * CHUNK
        for j in range(CHUNK // 16):
            d = pl.ds(16 * j, 16)
            sl = pl.ds(off + 16 * j, 16)
            rows_v[d] = rblk[sl]
            cidx[d] = cblk[sl]
            rvals[d] = vblk[sl]
        pltpu.async_copy(xs_hbm.at[cidx, pl.ds(pl.multiple_of(c * DH, 128), DH)], buf, gsem)

    def process(k, cidx, rows_v, rvals, buf, gsem, ssem):
        pltpu.make_async_copy(xs_hbm.at[cidx, pl.ds(pl.multiple_of(c * DH, 128), DH)], buf, gsem).wait()

        @pl.loop(0, CHUNK // 16)
        def _(g):
            v16 = rvals[pl.ds(g * 16, 16)]
            for el in range(16):
                valv = jnp.full((16,), v16[el], jnp.float32)
                row = buf.at[g * 16 + el]
                for j in range(DH // 16):
                    row[pl.ds(16 * j, 16)] = row[pl.ds(16 * j, 16)] * valv

        pltpu.async_copy(buf, acc.at[rows_v], ssem, add=True)

    sets = [
        (cidx0, rows0, rvals0, buf0, gsem0, ssem0),
        (cidx1, rows1, rvals1, buf1, gsem1, ssem1),
        (cidx2, rows2, rvals2, buf2, gsem2, ssem2),
        (cidx3, rows3, rvals3, buf3, gsem3, ssem3),
    ]

    start_block_load(0, 0)
    issue(0, *sets[0], True)
    issue(1, *sets[1], True)
    issue(2, *sets[2], True)
    process(0, *sets[0])
    issue(3, *sets[3], True)
    process(1, *sets[1])
    issue(4, *sets[0], False)
    process(2, *sets[2])
    issue(5, *sets[1], False)
    process(3, *sets[3])

    n_main = (N_CHUNKS - 1) // NBUF

    @pl.loop(1, n_main)
    def _(i):
        for j in range(NBUF):
            k = NBUF * i + j
            if j < NBUF - 1:
                issue(k + 2, *sets[(j + 2) % NBUF], False)
            else:
                @pl.when(i < n_main - 1)
                def _():
                    issue(k + 2, *sets[(j + 2) % NBUF], False)
            process(k, *sets[j])

    process(N_CHUNKS - 1, *sets[(N_CHUNKS - 1) % NBUF])

    for j in range(NBUF):
        cidx, rows_v, rvals, buf, gsem, ssem = sets[j]
        pltpu.make_async_copy(buf, acc.at[rows_v], ssem).wait()

    plsc.subcore_barrier()

    @pl.loop(0, OUT_TILE // CHUNK)
    def _(i):
        r0 = s * OUT_TILE + i * CHUNK
        pltpu.sync_copy(acc.at[pl.ds(r0, CHUNK)],
                        h_hbm.at[pl.ds(c * N_NODES + r0, CHUNK)])

    r1 = s * OUT_TILE + OUT_TILE - OUT_REM
    pltpu.sync_copy(acc.at[pl.ds(r1, OUT_REM)],
                    h_hbm.at[pl.ds(c * N_NODES + r1, OUT_REM)])

    @pl.when(s == 0)
    def _():
        r2 = 16 * OUT_TILE
        pltpu.sync_copy(acc.at[pl.ds(r2, OUT_TAIL)],
                        h_hbm.at[pl.ds(c * N_NODES + r2, OUT_TAIL)])


def _chunk_scratch():
    return [
        pltpu.VMEM((CHUNK,), jnp.int32),
        pltpu.VMEM((CHUNK,), jnp.int32),
        pltpu.VMEM((CHUNK,), jnp.float32),
        pltpu.VMEM((CHUNK, DH), jnp.float32),
    ]


def _spmm(rows, cols, vals, xs):
    mesh = plsc.VectorSubcoreMesh(core_axis_name="c", subcore_axis_name="s",
                                  num_cores=2, num_subcores=16)
    return pl.kernel(
        _spmm_body,
        out_type=jax.ShapeDtypeStruct((2 * N_NODES, DH), jnp.float32),
        mesh=mesh,
        scratch_types=(
            [
                pltpu.VMEM((2 * BLK_PAD,), jnp.int32),
                pltpu.VMEM((2 * BLK_PAD,), jnp.int32),
                pltpu.VMEM((2 * BLK_PAD,), jnp.float32),
            ]
            + _chunk_scratch() + _chunk_scratch()
            + _chunk_scratch() + _chunk_scratch()
            + [pltpu.VMEM_SHARED((N_NODES, DH), jnp.float32)]
            + [pltpu.SemaphoreType.DMA] * 9
        ),
        name="spmm_sc",
        compiler_params=pltpu.CompilerParams(needs_layout_passes=False),
    )(rows, cols, vals, xs)


BM = 2000


def _linear_body(h0_ref, h1_ref, a_ref, b_ref, o_ref):
    acc = jnp.dot(h0_ref[...], a_ref[0:DH, :],
                  preferred_element_type=jnp.float32)
    acc = acc + jnp.dot(h1_ref[...], a_ref[DH:D, :],
                        preferred_element_type=jnp.float32)
    o_ref[...] = jnp.maximum(acc + b_ref[...], 0.0)


def _linear_relu(h_stacked, a, b2):
    nb = N_NODES // BM
    return pl.pallas_call(
        _linear_body,
        grid=(nb,),
        in_specs=[
            pl.BlockSpec((BM, DH), lambda i: (i, 0)),
            pl.BlockSpec((BM, DH), lambda i, nb=nb: (i + nb, 0)),
            pl.BlockSpec((D, D), lambda i: (0, 0)),
            pl.BlockSpec((1, D), lambda i: (0, 0)),
        ],
        out_specs=pl.BlockSpec((BM, D), lambda i: (i, 0)),
        out_shape=jax.ShapeDtypeStruct((N_NODES, D), jnp.float32),
    )(h_stacked, h_stacked, a, b2)


@jax.jit
def kernel(x, adj_indices, adj_values, W, b):
    rows = adj_indices[0].astype(jnp.int32)
    cols = adj_indices[1].astype(jnp.int32)
    h_stacked = _spmm(rows, cols, adj_values, x)
    return _linear_relu(h_stacked, W.T, b.reshape(1, D))


# speedup vs baseline: 8.4090x; 1.0060x over previous
import jax
import jax.numpy as jnp
from jax import lax
from jax.experimental import pallas as pl
from jax.experimental.pallas import tpu as pltpu
from jax.experimental.pallas import tpu_sc as plsc

N_NODES = 10000
N_EDGES = 160000
D = 256
DH = D // 2

CHUNK = 80
EDGES_PER_TILE = N_EDGES // 16
N_CHUNKS = EDGES_PER_TILE // CHUNK
BLOCK = 5
N_BLOCKS = N_CHUNKS // BLOCK
BLK_LEN = BLOCK * CHUNK
BLK_PAD = 512
NBUF = 4
OUT_TILE = 624
OUT_TAIL = N_NODES - 16 * OUT_TILE
OUT_REM = OUT_TILE - (OUT_TILE // CHUNK) * CHUNK


def _spmm_body(rows_hbm, cols_hbm, vals_hbm, xs_hbm, h_hbm,
               rblk, cblk, vblk,
               cidx0, rows0, rvals0, buf0,
               cidx1, rows1, rvals1, buf1,
               cidx2, rows2, rvals2, buf2,
               cidx3, rows3, rvals3, buf3,
               acc,
               bsem, gsem0, gsem1, gsem2, gsem3,
               ssem0, ssem1, ssem2, ssem3):
    c = lax.axis_index("c")
    s = lax.axis_index("s")

    pltpu.async_copy(rows_hbm.at[pl.ds(s * EDGES_PER_TILE, BLK_LEN)],
                     rblk.at[pl.ds(0, BLK_LEN)], bsem)
    pltpu.async_copy(cols_hbm.at[pl.ds(s * EDGES_PER_TILE, BLK_LEN)],
                     cblk.at[pl.ds(0, BLK_LEN)], bsem)
    pltpu.async_copy(vals_hbm.at[pl.ds(s * EDGES_PER_TILE, BLK_LEN)],
                     vblk.at[pl.ds(0, BLK_LEN)], bsem)

    @pl.loop(0, CHUNK)
    def _(r):
        zrow = buf0.at[r]
        for j in range(DH // 16):
            zrow[pl.ds(16 * j, 16)] = jnp.zeros((16,), jnp.float32)

    @pl.loop(0, OUT_TILE // CHUNK)
    def _(i):
        pltpu.sync_copy(buf0, acc.at[pl.ds(s * OUT_TILE + i * CHUNK, CHUNK)])

    pltpu.sync_copy(buf0.at[pl.ds(0, OUT_REM)],
                    acc.at[pl.ds(s * OUT_TILE + OUT_TILE - OUT_REM, OUT_REM)])

    @pl.when(s == 0)
    def _():
        pltpu.sync_copy(buf0.at[pl.ds(0, OUT_TAIL)],
                        acc.at[pl.ds(16 * OUT_TILE, OUT_TAIL)])

    plsc.subcore_barrier()

    pass

    def start_block_load(b, slot):
        base = s * EDGES_PER_TILE + b * BLK_LEN
        dst = pl.ds(slot * BLK_PAD, BLK_LEN)
        pltpu.async_copy(rows_hbm.at[pl.ds(base, BLK_LEN)], rblk.at[dst], bsem)
        pltpu.async_copy(cols_hbm.at[pl.ds(base, BLK_LEN)], cblk.at[dst], bsem)
        pltpu.async_copy(vals_hbm.at[pl.ds(base, BLK_LEN)], vblk.at[dst], bsem)

    def wait_block(b, slot):
        base = s * EDGES_PER_TILE + b * BLK_LEN
        dst = pl.ds(slot * BLK_PAD, BLK_LEN)
        pltpu.make_async_copy(rows_hbm.at[pl.ds(base, BLK_LEN)], rblk.at[dst],
                              bsem).wait()
        pltpu.make_async_copy(cols_hbm.at[pl.ds(base, BLK_LEN)], cblk.at[dst],
                              bsem).wait()
        pltpu.make_async_copy(vals_hbm.at[pl.ds(base, BLK_LEN)], vblk.at[dst],
                              bsem).wait()

    def issue(k, cidx, rows_v, rvals, buf, gsem, ssem, first):
        if not first:
            pltpu.make_async_copy(buf, acc.at[rows_v], ssem).wait()
        b = k // BLOCK
        sub = k % BLOCK
        p = b % 2

        def rotate_blocks():
            wait_block(b, p)
            if isinstance(b, int):
                if b < N_BLOCKS - 1:
                    start_block_load(b + 1, 1 - p)
            else:
                @pl.when(b < N_BLOCKS - 1)
                def _():
                    start_block_load(b + 1, 1 - p)

        if isinstance(k, int):
            if sub == 0:
                rotate_blocks()
        else:
            @pl.when(sub == 0)
            def _():
                rotate_blocks()

        off = p * BLK_PAD + sub * CHUNK
        for j in range(CHUNK // 16):
            d = pl.ds(16 * j, 16)
            sl = pl.ds(off + 16 * j, 16)
            rows_v[d] = rblk[sl]
            cidx[d] = cblk[sl]
            rvals[d] = vblk[sl]
        pltpu.async_copy(xs_hbm.at[cidx, pl.ds(pl.multiple_of(c * DH, 128), DH)], buf, gsem)

    def process(k, cidx, rows_v, rvals, buf, gsem, ssem):
        pltpu.make_async_copy(xs_hbm.at[cidx, pl.ds(pl.multiple_of(c * DH, 128), DH)], buf, gsem).wait()

        @pl.loop(0, CHUNK // 16)
        def _(g):
            v16 = rvals[pl.ds(g * 16, 16)]
            for el in range(16):
                valv = jnp.full((16,), v16[el], jnp.float32)
                row = buf.at[g * 16 + el]
                for j in range(DH // 16):
                    row[pl.ds(16 * j, 16)] = row[pl.ds(16 * j, 16)] * valv

        pltpu.async_copy(buf, acc.at[rows_v], ssem, add=True)

    sets = [
        (cidx0, rows0, rvals0, buf0, gsem0, ssem0),
        (cidx1, rows1, rvals1, buf1, gsem1, ssem1),
        (cidx2, rows2, rvals2, buf2, gsem2, ssem2),
        (cidx3, rows3, rvals3, buf3, gsem3, ssem3),
    ]

    issue(0, *sets[0], True)
    issue(1, *sets[1], True)
    issue(2, *sets[2], True)
    process(0, *sets[0])
    issue(3, *sets[3], True)
    process(1, *sets[1])
    issue(4, *sets[0], False)
    process(2, *sets[2])
    issue(5, *sets[1], False)
    process(3, *sets[3])

    n_main = (N_CHUNKS - 1) // NBUF

    @pl.loop(1, n_main)
    def _(i):
        for j in range(NBUF):
            k = NBUF * i + j
            if j < NBUF - 1:
                issue(k + 2, *sets[(j + 2) % NBUF], False)
            else:
                @pl.when(i < n_main - 1)
                def _():
                    issue(k + 2, *sets[(j + 2) % NBUF], False)
            process(k, *sets[j])

    process(N_CHUNKS - 1, *sets[(N_CHUNKS - 1) % NBUF])

    for j in range(NBUF):
        cidx, rows_v, rvals, buf, gsem, ssem = sets[j]
        pltpu.make_async_copy(buf, acc.at[rows_v], ssem).wait()

    plsc.subcore_barrier()

    @pl.loop(0, OUT_TILE // CHUNK)
    def _(i):
        r0 = s * OUT_TILE + i * CHUNK
        pltpu.sync_copy(acc.at[pl.ds(r0, CHUNK)],
                        h_hbm.at[pl.ds(c * N_NODES + r0, CHUNK)])

    r1 = s * OUT_TILE + OUT_TILE - OUT_REM
    pltpu.sync_copy(acc.at[pl.ds(r1, OUT_REM)],
                    h_hbm.at[pl.ds(c * N_NODES + r1, OUT_REM)])

    @pl.when(s == 0)
    def _():
        r2 = 16 * OUT_TILE
        pltpu.sync_copy(acc.at[pl.ds(r2, OUT_TAIL)],
                        h_hbm.at[pl.ds(c * N_NODES + r2, OUT_TAIL)])


def _chunk_scratch():
    return [
        pltpu.VMEM((CHUNK,), jnp.int32),
        pltpu.VMEM((CHUNK,), jnp.int32),
        pltpu.VMEM((CHUNK,), jnp.float32),
        pltpu.VMEM((CHUNK, DH), jnp.float32),
    ]


def _spmm(rows, cols, vals, xs):
    mesh = plsc.VectorSubcoreMesh(core_axis_name="c", subcore_axis_name="s",
                                  num_cores=2, num_subcores=16)
    return pl.kernel(
        _spmm_body,
        out_type=jax.ShapeDtypeStruct((2 * N_NODES, DH), jnp.float32),
        mesh=mesh,
        scratch_types=(
            [
                pltpu.VMEM((2 * BLK_PAD,), jnp.int32),
                pltpu.VMEM((2 * BLK_PAD,), jnp.int32),
                pltpu.VMEM((2 * BLK_PAD,), jnp.float32),
            ]
            + _chunk_scratch() + _chunk_scratch()
            + _chunk_scratch() + _chunk_scratch()
            + [pltpu.VMEM_SHARED((N_NODES, DH), jnp.float32)]
            + [pltpu.SemaphoreType.DMA] * 9
        ),
        name="spmm_sc",
        compiler_params=pltpu.CompilerParams(needs_layout_passes=False),
    )(rows, cols, vals, xs)


BM = 2000


_DN = (((1,), (1,)), ((), ()))


def _linear_body(h0_ref, h1_ref, w_ref, b_ref, o_ref):
    acc = lax.dot_general(h0_ref[...], w_ref[:, 0:DH], _DN,
                          preferred_element_type=jnp.float32)
    acc = acc + lax.dot_general(h1_ref[...], w_ref[:, DH:D], _DN,
                                preferred_element_type=jnp.float32)
    o_ref[...] = jnp.maximum(acc + b_ref[...], 0.0)


def _linear_relu(h_stacked, a, b2):
    nb = N_NODES // BM
    return pl.pallas_call(
        _linear_body,
        grid=(nb,),
        in_specs=[
            pl.BlockSpec((BM, DH), lambda i: (i, 0)),
            pl.BlockSpec((BM, DH), lambda i, nb=nb: (i + nb, 0)),
            pl.BlockSpec((D, D), lambda i: (0, 0)),
            pl.BlockSpec((1, D), lambda i: (0, 0)),
        ],
        out_specs=pl.BlockSpec((BM, D), lambda i: (i, 0)),
        out_shape=jax.ShapeDtypeStruct((N_NODES, D), jnp.float32),
    )(h_stacked, h_stacked, a, b2)


@jax.jit
def kernel(x, adj_indices, adj_values, W, b):
    rows = adj_indices[0].astype(jnp.int32)
    cols = adj_indices[1].astype(jnp.int32)
    h_stacked = _spmm(rows, cols, adj_values, x)
    return _linear_relu(h_stacked, W, b.reshape(1, D))
